# Initial kernel scaffold; baseline (speedup 1.0000x reference)
import jax
import jax.numpy as jnp
from jax.experimental import pallas as pl


def kernel(x, edge_index, batch, static_features, W1, b1, W2, b2, W3, b3, Wl, bl, Wf1, bf1, Wf2, bf2, Wf3, bf3, Wf4, bf4, Wf5, bf5):
    raise NotImplementedError("write your pallas kernel here")


# speedup vs baseline: 10.8007x; 10.8007x over previous
import functools

import jax
import jax.numpy as jnp
from jax import lax
from jax.experimental import pallas as pl
from jax.experimental.pallas import tpu as pltpu
from jax.experimental.pallas import tpu_sc as plsc

N = 10000
E = 320000
D = 128
H = 256
G = 64
S = 16

NC, NS = 2, 16
NP = 10240
RT = NP // NS
K = 80
NCHUNK = E // K
CH_A = NCHUNK // NS
CH_D = NCHUNK // (NC * NS)
SUB = 5
RSUB = RT // SUB
HH = H // 2
BM = 1024
NBLK = NP // BM

_f32 = jnp.float32

_sc_mesh = plsc.VectorSubcoreMesh(
    core_axis_name="c", subcore_axis_name="s", num_cores=NC, num_subcores=NS)


DW = 128


def _deg_body(dst3, zeros_h, ones_h, dd, idx_blk, ones_v, acc):
  c = lax.axis_index("c")
  s = lax.axis_index("s")
  rows = pl.ds(s * RT, RT)
  pltpu.sync_copy(zeros_h, acc.at[rows])
  pltpu.sync_copy(ones_h, ones_v)
  tid = c * NS + s
  pltpu.sync_copy(dst3.at[tid], idx_blk)
  plsc.subcore_barrier()

  def body(j, carry):
    pltpu.sync_copy(ones_v, acc.at[idx_blk.at[j]], add=True)
    return carry

  lax.fori_loop(0, CH_D, body, 0)
  plsc.subcore_barrier()
  pltpu.sync_copy(acc.at[rows], dd.at[pl.ds(c * NP + s * RT, RT)])


_deg_call = pl.kernel(
    _deg_body,
    out_type=jax.ShapeDtypeStruct((NC * NP, DW), _f32),
    mesh=_sc_mesh,
    scratch_types=[
        pltpu.VMEM((CH_D, K), jnp.int32),
        pltpu.VMEM((K, DW), _f32),
        pltpu.VMEM_SHARED((NP, DW), _f32),
    ],
)


def _dinv_kernel(d0_ref, d1_ref, o_ref):
  o_ref[...] = lax.rsqrt(
      d0_ref[...][:, 0:1] + d1_ref[...][:, 0:1] + 1.0)


def _dinv_call(d0, d1):
  return pl.pallas_call(
      _dinv_kernel,
      out_shape=jax.ShapeDtypeStruct((NP, 1), _f32),
  )(d0, d1)


def _mm_kernel(pre, *refs):
  if pre:
    a0, a1, w0, w1, dinv_ref, bp_ref, y0_ref, y1_ref = refs
    dinv = dinv_ref[...]
    bp = bp_ref[...]
    h0 = jnp.maximum(a0[...] * dinv + bp[:, :HH], 0.0)
    h1 = jnp.maximum(a1[...] * dinv + bp[:, HH:], 0.0)
    acc = jnp.dot(h0, w0[...], preferred_element_type=_f32)
    acc += jnp.dot(h1, w1[...], preferred_element_type=_f32)
  else:
    h0, w0, dinv_ref, y0_ref, y1_ref = refs
    dinv = dinv_ref[...]
    acc = jnp.dot(h0[...], w0[...], preferred_element_type=_f32)
  y = acc * dinv
  y0_ref[...] = y[:, :HH]
  y1_ref[...] = y[:, HH:]


def _mm_call(h_parts, w_parts, dinv_n1, b_prev=None):
  nparts = len(h_parts)
  din = h_parts[0].shape[1]
  pre = b_prev is not None
  in_specs = (
      [pl.BlockSpec((BM, din), lambda i: (i, 0)) for _ in range(nparts)] +
      [pl.BlockSpec((din, H), lambda i: (0, 0)) for _ in range(nparts)] +
      [pl.BlockSpec((BM, 1), lambda i: (i, 0))])
  args = list(h_parts) + list(w_parts) + [dinv_n1]
  if pre:
    in_specs.append(pl.BlockSpec((1, H), lambda i: (0, 0)))
    args.append(b_prev)
  out_specs = (pl.BlockSpec((BM, HH), lambda i: (i, 0)),
               pl.BlockSpec((BM, HH), lambda i: (i, 0)))
  return pl.pallas_call(
      functools.partial(_mm_kernel, pre),
      grid=(NBLK,),
      in_specs=in_specs,
      out_specs=out_specs,
      out_shape=(jax.ShapeDtypeStruct((NP, HH), _f32),
                 jax.ShapeDtypeStruct((NP, HH), _f32)),
  )(*args)


def _agg_half(y2, out2, s, src4, dst4, src_blk, dst_blk, rows_v, sem, acc):
  mine = pl.ds(s * RT, RT)
  pltpu.sync_copy(y2.at[mine], acc.at[mine])
  plsc.subcore_barrier()

  for half in range(CH_A // CH_D):
    pltpu.sync_copy(src4.at[s, half], src_blk)
    pltpu.sync_copy(dst4.at[s, half], dst_blk)

    def edge(j, carry):
      pltpu.async_copy(y2.at[src_blk.at[j]], rows_v, sem).wait()
      pltpu.sync_copy(rows_v, acc.at[dst_blk.at[j]], add=True)
      return carry

    lax.fori_loop(0, CH_D, edge, 0)

  plsc.subcore_barrier()
  pltpu.sync_copy(acc.at[mine], out2.at[mine])


def _agg_body(y0, y1, src4, dst4, a0, a1,
              src_blk, dst_blk, rows_v, sem, acc):
  c = lax.axis_index("c")
  s = lax.axis_index("s")

  @pl.when(c == 0)
  def _():
    _agg_half(y0, a0, s, src4, dst4, src_blk, dst_blk, rows_v, sem, acc)

  @pl.when(c == 1)
  def _():
    _agg_half(y1, a1, s, src4, dst4, src_blk, dst_blk, rows_v, sem, acc)


_agg_call = pl.kernel(
    _agg_body,
    out_type=(jax.ShapeDtypeStruct((NP, HH), _f32),
              jax.ShapeDtypeStruct((NP, HH), _f32)),
    mesh=_sc_mesh,
    scratch_types=[
        pltpu.VMEM((CH_D, K), jnp.int32),
        pltpu.VMEM((CH_D, K), jnp.int32),
        pltpu.VMEM((K, HH), _f32),
        pltpu.SemaphoreType.DMA,
        pltpu.VMEM_SHARED((NP, HH), _f32),
    ],
)


def _pool_head_kernel(a0_ref, a1_ref, dinv_ref, b3_ref, batch_ref, st_ref,
                      wl_ref, bl_ref,
                      wf1a_ref, wf1b_ref, bf1_ref, wf2_ref, bf2_ref,
                      wf3_ref, bf3_ref, wf4_ref, bf4_ref, wf5_ref, bf5_ref,
                      o_ref, acc_p, acc_c):
  i = pl.program_id(0)

  @pl.when(i == 0)
  def _():
    acc_p[...] = jnp.zeros_like(acc_p)
    acc_c[...] = jnp.zeros_like(acc_c)

  dinv = dinv_ref[...]
  b3 = b3_ref[...]
  g0 = a0_ref[...] * dinv + b3[:, :HH]
  g1 = a1_ref[...] * dinv + b3[:, HH:]
  ids = lax.broadcasted_iota(jnp.int32, (1, G), 1)
  p = (batch_ref[...] == ids).astype(_f32)
  cdims = (((0,), (0,)), ((), ()))
  acc_p[:, :HH] += lax.dot_general(p, g0, cdims,
                                   preferred_element_type=_f32, precision=lax.Precision.HIGHEST)
  acc_p[:, HH:] += lax.dot_general(p, g1, cdims,
                                   preferred_element_type=_f32, precision=lax.Precision.HIGHEST)
  acc_c[...] += lax.dot_general(p, jnp.ones((BM, 1), _f32), cdims,
                                preferred_element_type=_f32, precision=lax.Precision.HIGHEST)

  @pl.when(i == NBLK - 1)
  def _():
    pooled = acc_p[...] / jnp.maximum(acc_c[...], 1.0)
    z = jnp.dot(pooled, wl_ref[...], preferred_element_type=_f32, precision=lax.Precision.HIGHEST) + bl_ref[...]
    zr = jnp.maximum(z, 0.0)
    sr = jnp.maximum(st_ref[...], 0.0)
    z = jnp.dot(zr, wf1a_ref[...], preferred_element_type=_f32, precision=lax.Precision.HIGHEST)
    z += jnp.dot(sr, wf1b_ref[...], preferred_element_type=_f32, precision=lax.Precision.HIGHEST)
    z = jnp.maximum(z + bf1_ref[...], 0.0)
    z = jnp.maximum(
        jnp.dot(z, wf2_ref[...], preferred_element_type=_f32, precision=lax.Precision.HIGHEST) + bf2_ref[...],
        0.0)
    z = jnp.maximum(
        jnp.dot(z, wf3_ref[...], preferred_element_type=_f32, precision=lax.Precision.HIGHEST) + bf3_ref[...],
        0.0)
    z = jnp.maximum(
        jnp.dot(z, wf4_ref[...], preferred_element_type=_f32, precision=lax.Precision.HIGHEST) + bf4_ref[...],
        0.0)
    o_ref[...] = (
        jnp.dot(z, wf5_ref[...], preferred_element_type=_f32, precision=lax.Precision.HIGHEST) + bf5_ref[...])


def _pool_head_call(a0, a1, dinv_n1, b3, batch_n1, st, wl, bl, wf1a, wf1b,
                    bf1, wf2, bf2, wf3, bf3, wf4, bf4, wf5, bf5):
  full = lambda a: pl.BlockSpec(a.shape, lambda i: tuple(0 for _ in a.shape))
  in_specs = [
      pl.BlockSpec((BM, HH), lambda i: (i, 0)),
      pl.BlockSpec((BM, HH), lambda i: (i, 0)),
      pl.BlockSpec((BM, 1), lambda i: (i, 0)),
      pl.BlockSpec((1, H), lambda i: (0, 0)),
      pl.BlockSpec((BM, 1), lambda i: (i, 0)),
      full(st), full(wl), full(bl), full(wf1a), full(wf1b), full(bf1),
      full(wf2), full(bf2), full(wf3), full(bf3), full(wf4), full(bf4),
      full(wf5), full(bf5),
  ]
  return pl.pallas_call(
      _pool_head_kernel,
      grid=(NBLK,),
      in_specs=in_specs,
      out_specs=pl.BlockSpec((G, 1), lambda i: (0, 0)),
      out_shape=jax.ShapeDtypeStruct((G, 1), _f32),
      scratch_shapes=[pltpu.VMEM((G, H), _f32), pltpu.VMEM((G, 1), _f32)],
  )(a0, a1, dinv_n1, b3, batch_n1, st, wl, bl, wf1a, wf1b, bf1, wf2, bf2,
    wf3, bf3, wf4, bf4, wf5, bf5)


def kernel(x, edge_index, batch, static_features, W1, b1, W2, b2, W3, b3,
           Wl, bl, Wf1, bf1, Wf2, bf2, Wf3, bf3, Wf4, bf4, Wf5, bf5):
  src3 = edge_index[0].reshape(NS, CH_A // CH_D, CH_D, K)
  dst3 = edge_index[1].reshape(NS, CH_A // CH_D, CH_D, K)
  dst3d = edge_index[1].reshape(NC * NS, CH_D, K)
  zeros_h = jnp.zeros((RT, DW), _f32)
  ones_h = jnp.ones((K, DW), _f32)
  x_p = jnp.pad(x, ((0, NP - N), (0, 0)))
  batch_p = jnp.pad(batch, (0, NP - N), constant_values=G)

  dd = _deg_call(dst3d, zeros_h, ones_h)
  dinv_n1 = _dinv_call(dd[:NP], dd[NP:])

  y0, y1 = _mm_call([x_p], [W1], dinv_n1)
  a0, a1 = _agg_call(y0, y1, src3, dst3)

  y0, y1 = _mm_call([a0, a1], [W2[:HH], W2[HH:]], dinv_n1,
                    b_prev=b1.reshape(1, H))
  a0, a1 = _agg_call(y0, y1, src3, dst3)

  y0, y1 = _mm_call([a0, a1], [W3[:HH], W3[HH:]], dinv_n1,
                    b_prev=b2.reshape(1, H))
  a0, a1 = _agg_call(y0, y1, src3, dst3)

  out = _pool_head_call(
      a0, a1, dinv_n1, b3.reshape(1, H), batch_p.reshape(NP, 1),
      static_features, Wl, bl.reshape(1, 9),
      Wf1[:9], Wf1[9:], bf1.reshape(1, 32), Wf2, bf2.reshape(1, 64),
      Wf3, bf3.reshape(1, 32), Wf4, bf4.reshape(1, 16), Wf5,
      bf5.reshape(1, 1))
  return out.reshape(G)


# speedup vs baseline: 13.1969x; 1.2219x over previous
import functools

import jax
import jax.numpy as jnp
from jax import lax
from jax.experimental import pallas as pl
from jax.experimental.pallas import tpu as pltpu
from jax.experimental.pallas import tpu_sc as plsc

N = 10000
E = 320000
D = 128
H = 256
G = 64
S = 16

NC, NS = 2, 16
NP = 10240
RT = NP // NS
K = 80
NCHUNK = E // K
CH_A = NCHUNK // NS
CH_D = NCHUNK // (NC * NS)
SUB = 5
RSUB = RT // SUB
HH = H // 2
BM = 1024
NBLK = NP // BM

_f32 = jnp.float32

_sc_mesh = plsc.VectorSubcoreMesh(
    core_axis_name="c", subcore_axis_name="s", num_cores=NC, num_subcores=NS)


DW = 128


def _deg_body(dst3, zeros_h, ones_h, dd, idx_blk, ones_v, acc):
  c = lax.axis_index("c")
  s = lax.axis_index("s")
  rows = pl.ds(s * RT, RT)
  pltpu.sync_copy(zeros_h, acc.at[rows])
  pltpu.sync_copy(ones_h, ones_v)
  tid = c * NS + s
  pltpu.sync_copy(dst3.at[tid], idx_blk)
  plsc.subcore_barrier()

  def body(j, carry):
    pltpu.sync_copy(ones_v, acc.at[idx_blk.at[j]], add=True)
    return carry

  lax.fori_loop(0, CH_D, body, 0)
  plsc.subcore_barrier()
  pltpu.sync_copy(acc.at[rows], dd.at[pl.ds(c * NP + s * RT, RT)])


_deg_call = pl.kernel(
    _deg_body,
    out_type=jax.ShapeDtypeStruct((NC * NP, DW), _f32),
    mesh=_sc_mesh,
    scratch_types=[
        pltpu.VMEM((CH_D, K), jnp.int32),
        pltpu.VMEM((K, DW), _f32),
        pltpu.VMEM_SHARED((NP, DW), _f32),
    ],
)


def _dinv_kernel(d0_ref, d1_ref, o_ref):
  o_ref[...] = lax.rsqrt(
      d0_ref[...][:, 0:1] + d1_ref[...][:, 0:1] + 1.0)


def _dinv_call(d0, d1):
  return pl.pallas_call(
      _dinv_kernel,
      out_shape=jax.ShapeDtypeStruct((NP, 1), _f32),
  )(d0, d1)


def _mm_kernel(pre, *refs):
  if pre:
    a0, a1, w0, w1, dinv_ref, bp_ref, y0_ref, y1_ref = refs
    dinv = dinv_ref[...]
    bp = bp_ref[...]
    h0 = jnp.maximum(a0[...] * dinv + bp[:, :HH], 0.0)
    h1 = jnp.maximum(a1[...] * dinv + bp[:, HH:], 0.0)
    acc = jnp.dot(h0, w0[...], preferred_element_type=_f32)
    acc += jnp.dot(h1, w1[...], preferred_element_type=_f32)
  else:
    h0, w0, dinv_ref, y0_ref, y1_ref = refs
    dinv = dinv_ref[...]
    acc = jnp.dot(h0[...], w0[...], preferred_element_type=_f32)
  y = acc * dinv
  y0_ref[...] = y[:, :HH]
  y1_ref[...] = y[:, HH:]


def _mm_call(h_parts, w_parts, dinv_n1, b_prev=None):
  nparts = len(h_parts)
  din = h_parts[0].shape[1]
  pre = b_prev is not None
  in_specs = (
      [pl.BlockSpec((BM, din), lambda i: (i, 0)) for _ in range(nparts)] +
      [pl.BlockSpec((din, H), lambda i: (0, 0)) for _ in range(nparts)] +
      [pl.BlockSpec((BM, 1), lambda i: (i, 0))])
  args = list(h_parts) + list(w_parts) + [dinv_n1]
  if pre:
    in_specs.append(pl.BlockSpec((1, H), lambda i: (0, 0)))
    args.append(b_prev)
  out_specs = (pl.BlockSpec((BM, HH), lambda i: (i, 0)),
               pl.BlockSpec((BM, HH), lambda i: (i, 0)))
  return pl.pallas_call(
      functools.partial(_mm_kernel, pre),
      grid=(NBLK,),
      in_specs=in_specs,
      out_specs=out_specs,
      out_shape=(jax.ShapeDtypeStruct((NP, HH), _f32),
                 jax.ShapeDtypeStruct((NP, HH), _f32)),
  )(*args)


GP = 25
NG = CH_D // GP


def _agg_half(y2, out2, s, src4, dst4, src_blk, dst_blk, r0, r1,
              gs0, gs1, ss0, ss1, acc):
  mine = pl.ds(s * RT, RT)
  pltpu.sync_copy(y2.at[mine], acc.at[mine])
  plsc.subcore_barrier()
  rows = (r0, r1)
  gsem = (gs0, gs1)
  ssem = (ss0, ss1)

  for half in range(CH_A // CH_D):

    def group(g, carry):
      pltpu.sync_copy(src4.at[s, half, g], src_blk)
      pltpu.sync_copy(dst4.at[s, half, g], dst_blk)
      gd = [None] * GP
      sd = [None] * GP
      gd[0] = pltpu.async_copy(y2.at[src_blk.at[0]], rows[0], gsem[0])
      for u in range(GP):
        b = u % 2
        gd[u].wait()
        if u >= 1:
          sd[u - 1].wait()
        if u < GP - 1:
          gd[u + 1] = pltpu.async_copy(
              y2.at[src_blk.at[u + 1]], rows[1 - b], gsem[1 - b])
        sd[u] = pltpu.async_copy(
            rows[b], acc.at[dst_blk.at[u]], ssem[b], add=True)
      sd[GP - 1].wait()
      return carry

    lax.fori_loop(0, NG, group, 0)

  plsc.subcore_barrier()
  pltpu.sync_copy(acc.at[mine], out2.at[mine])


def _agg_body(y0, y1, src4, dst4, a0, a1,
              src_blk, dst_blk, r0, r1, gs0, gs1, ss0, ss1, acc):
  c = lax.axis_index("c")
  s = lax.axis_index("s")

  @pl.when(c == 0)
  def _():
    _agg_half(y0, a0, s, src4, dst4, src_blk, dst_blk, r0, r1,
              gs0, gs1, ss0, ss1, acc)

  @pl.when(c == 1)
  def _():
    _agg_half(y1, a1, s, src4, dst4, src_blk, dst_blk, r0, r1,
              gs0, gs1, ss0, ss1, acc)


_agg_call = pl.kernel(
    _agg_body,
    out_type=(jax.ShapeDtypeStruct((NP, HH), _f32),
              jax.ShapeDtypeStruct((NP, HH), _f32)),
    mesh=_sc_mesh,
    scratch_types=[
        pltpu.VMEM((GP, K), jnp.int32),
        pltpu.VMEM((GP, K), jnp.int32),
        pltpu.VMEM((K, HH), _f32),
        pltpu.VMEM((K, HH), _f32),
        pltpu.SemaphoreType.DMA,
        pltpu.SemaphoreType.DMA,
        pltpu.SemaphoreType.DMA,
        pltpu.SemaphoreType.DMA,
        pltpu.VMEM_SHARED((NP, HH), _f32),
    ],
)


def _pool_head_kernel(a0_ref, a1_ref, dinv_ref, b3_ref, batch_ref, st_ref,
                      wl_ref, bl_ref,
                      wf1a_ref, wf1b_ref, bf1_ref, wf2_ref, bf2_ref,
                      wf3_ref, bf3_ref, wf4_ref, bf4_ref, wf5_ref, bf5_ref,
                      o_ref, acc_p, acc_c):
  i = pl.program_id(0)

  @pl.when(i == 0)
  def _():
    acc_p[...] = jnp.zeros_like(acc_p)
    acc_c[...] = jnp.zeros_like(acc_c)

  dinv = dinv_ref[...]
  b3 = b3_ref[...]
  g0 = a0_ref[...] * dinv + b3[:, :HH]
  g1 = a1_ref[...] * dinv + b3[:, HH:]
  ids = lax.broadcasted_iota(jnp.int32, (1, G), 1)
  p = (batch_ref[...] == ids).astype(_f32)
  cdims = (((0,), (0,)), ((), ()))
  acc_p[:, :HH] += lax.dot_general(p, g0, cdims,
                                   preferred_element_type=_f32, precision=lax.Precision.HIGHEST)
  acc_p[:, HH:] += lax.dot_general(p, g1, cdims,
                                   preferred_element_type=_f32, precision=lax.Precision.HIGHEST)
  acc_c[...] += lax.dot_general(p, jnp.ones((BM, 1), _f32), cdims,
                                preferred_element_type=_f32, precision=lax.Precision.HIGHEST)

  @pl.when(i == NBLK - 1)
  def _():
    pooled = acc_p[...] / jnp.maximum(acc_c[...], 1.0)
    z = jnp.dot(pooled, wl_ref[...], preferred_element_type=_f32, precision=lax.Precision.HIGHEST) + bl_ref[...]
    zr = jnp.maximum(z, 0.0)
    sr = jnp.maximum(st_ref[...], 0.0)
    z = jnp.dot(zr, wf1a_ref[...], preferred_element_type=_f32, precision=lax.Precision.HIGHEST)
    z += jnp.dot(sr, wf1b_ref[...], preferred_element_type=_f32, precision=lax.Precision.HIGHEST)
    z = jnp.maximum(z + bf1_ref[...], 0.0)
    z = jnp.maximum(
        jnp.dot(z, wf2_ref[...], preferred_element_type=_f32, precision=lax.Precision.HIGHEST) + bf2_ref[...],
        0.0)
    z = jnp.maximum(
        jnp.dot(z, wf3_ref[...], preferred_element_type=_f32, precision=lax.Precision.HIGHEST) + bf3_ref[...],
        0.0)
    z = jnp.maximum(
        jnp.dot(z, wf4_ref[...], preferred_element_type=_f32, precision=lax.Precision.HIGHEST) + bf4_ref[...],
        0.0)
    o_ref[...] = (
        jnp.dot(z, wf5_ref[...], preferred_element_type=_f32, precision=lax.Precision.HIGHEST) + bf5_ref[...])


def _pool_head_call(a0, a1, dinv_n1, b3, batch_n1, st, wl, bl, wf1a, wf1b,
                    bf1, wf2, bf2, wf3, bf3, wf4, bf4, wf5, bf5):
  full = lambda a: pl.BlockSpec(a.shape, lambda i: tuple(0 for _ in a.shape))
  in_specs = [
      pl.BlockSpec((BM, HH), lambda i: (i, 0)),
      pl.BlockSpec((BM, HH), lambda i: (i, 0)),
      pl.BlockSpec((BM, 1), lambda i: (i, 0)),
      pl.BlockSpec((1, H), lambda i: (0, 0)),
      pl.BlockSpec((BM, 1), lambda i: (i, 0)),
      full(st), full(wl), full(bl), full(wf1a), full(wf1b), full(bf1),
      full(wf2), full(bf2), full(wf3), full(bf3), full(wf4), full(bf4),
      full(wf5), full(bf5),
  ]
  return pl.pallas_call(
      _pool_head_kernel,
      grid=(NBLK,),
      in_specs=in_specs,
      out_specs=pl.BlockSpec((G, 1), lambda i: (0, 0)),
      out_shape=jax.ShapeDtypeStruct((G, 1), _f32),
      scratch_shapes=[pltpu.VMEM((G, H), _f32), pltpu.VMEM((G, 1), _f32)],
  )(a0, a1, dinv_n1, b3, batch_n1, st, wl, bl, wf1a, wf1b, bf1, wf2, bf2,
    wf3, bf3, wf4, bf4, wf5, bf5)


def kernel(x, edge_index, batch, static_features, W1, b1, W2, b2, W3, b3,
           Wl, bl, Wf1, bf1, Wf2, bf2, Wf3, bf3, Wf4, bf4, Wf5, bf5):
  src3 = edge_index[0].reshape(NS, CH_A // CH_D, NG, GP, K)
  dst3 = edge_index[1].reshape(NS, CH_A // CH_D, NG, GP, K)
  dst3d = edge_index[1].reshape(NC * NS, CH_D, K)
  zeros_h = jnp.zeros((RT, DW), _f32)
  ones_h = jnp.ones((K, DW), _f32)
  x_p = jnp.pad(x, ((0, NP - N), (0, 0)))
  batch_p = jnp.pad(batch, (0, NP - N), constant_values=G)

  dd = _deg_call(dst3d, zeros_h, ones_h)
  dinv_n1 = _dinv_call(dd[:NP], dd[NP:])

  y0, y1 = _mm_call([x_p], [W1], dinv_n1)
  a0, a1 = _agg_call(y0, y1, src3, dst3)

  y0, y1 = _mm_call([a0, a1], [W2[:HH], W2[HH:]], dinv_n1,
                    b_prev=b1.reshape(1, H))
  a0, a1 = _agg_call(y0, y1, src3, dst3)

  y0, y1 = _mm_call([a0, a1], [W3[:HH], W3[HH:]], dinv_n1,
                    b_prev=b2.reshape(1, H))
  a0, a1 = _agg_call(y0, y1, src3, dst3)

  out = _pool_head_call(
      a0, a1, dinv_n1, b3.reshape(1, H), batch_p.reshape(NP, 1),
      static_features, Wl, bl.reshape(1, 9),
      Wf1[:9], Wf1[9:], bf1.reshape(1, 32), Wf2, bf2.reshape(1, 64),
      Wf3, bf3.reshape(1, 32), Wf4, bf4.reshape(1, 16), Wf5,
      bf5.reshape(1, 1))
  return out.reshape(G)


# speedup vs baseline: 18.8773x; 1.4304x over previous
import functools

import jax
import jax.numpy as jnp
from jax import lax
from jax.experimental import pallas as pl
from jax.experimental.pallas import tpu as pltpu
from jax.experimental.pallas import tpu_sc as plsc

N = 10000
E = 320000
D = 128
H = 256
G = 64
S = 16

NC, NS = 2, 16
NP = 10240
RT = NP // NS
K = 80
NCHUNK = E // K
CH_A = NCHUNK // NS
CH_D = NCHUNK // (NC * NS)
SUB = 5
RSUB = RT // SUB
HH = H // 2
BM = 1024
NBLK = NP // BM

_f32 = jnp.float32

_sc_mesh = plsc.VectorSubcoreMesh(
    core_axis_name="c", subcore_axis_name="s", num_cores=NC, num_subcores=NS)


DW = 128


def _deg_body(dst3, zeros_h, ones_h, dd, idx_blk, ones_v, acc):
  c = lax.axis_index("c")
  s = lax.axis_index("s")
  rows = pl.ds(s * RT, RT)
  pltpu.sync_copy(zeros_h, acc.at[rows])
  pltpu.sync_copy(ones_h, ones_v)
  tid = c * NS + s
  pltpu.sync_copy(dst3.at[tid], idx_blk)
  plsc.subcore_barrier()

  def body(j, carry):
    pltpu.sync_copy(ones_v, acc.at[idx_blk.at[j]], add=True)
    return carry

  lax.fori_loop(0, CH_D, body, 0)
  plsc.subcore_barrier()
  pltpu.sync_copy(acc.at[rows], dd.at[pl.ds(c * NP + s * RT, RT)])


_deg_call = pl.kernel(
    _deg_body,
    out_type=jax.ShapeDtypeStruct((NC * NP, DW), _f32),
    mesh=_sc_mesh,
    scratch_types=[
        pltpu.VMEM((CH_D, K), jnp.int32),
        pltpu.VMEM((K, DW), _f32),
        pltpu.VMEM_SHARED((NP, DW), _f32),
    ],
)


def _dinv_kernel(d0_ref, d1_ref, o_ref):
  o_ref[...] = lax.rsqrt(
      d0_ref[...][:, 0:1] + d1_ref[...][:, 0:1] + 1.0)


def _dinv_call(d0, d1):
  return pl.pallas_call(
      _dinv_kernel,
      out_shape=jax.ShapeDtypeStruct((NP, 1), _f32),
  )(d0, d1)


def _mm_kernel(pre, *refs):
  if pre:
    a0, a1, w0, w1, dinv_ref, bp_ref, y0_ref, y1_ref = refs
    dinv = dinv_ref[...]
    bp = bp_ref[...]
    h0 = jnp.maximum(a0[...] * dinv + bp[:, :HH], 0.0)
    h1 = jnp.maximum(a1[...] * dinv + bp[:, HH:], 0.0)
    acc = jnp.dot(h0, w0[...], preferred_element_type=_f32)
    acc += jnp.dot(h1, w1[...], preferred_element_type=_f32)
  else:
    h0, w0, dinv_ref, y0_ref, y1_ref = refs
    dinv = dinv_ref[...]
    acc = jnp.dot(h0[...], w0[...], preferred_element_type=_f32)
  y = acc * dinv
  y0_ref[...] = y[:, :HH]
  y1_ref[...] = y[:, HH:]


def _mm_call(h_parts, w_parts, dinv_n1, b_prev=None):
  nparts = len(h_parts)
  din = h_parts[0].shape[1]
  pre = b_prev is not None
  in_specs = (
      [pl.BlockSpec((BM, din), lambda i: (i, 0)) for _ in range(nparts)] +
      [pl.BlockSpec((din, H), lambda i: (0, 0)) for _ in range(nparts)] +
      [pl.BlockSpec((BM, 1), lambda i: (i, 0))])
  args = list(h_parts) + list(w_parts) + [dinv_n1]
  if pre:
    in_specs.append(pl.BlockSpec((1, H), lambda i: (0, 0)))
    args.append(b_prev)
  out_specs = (pl.BlockSpec((BM, HH), lambda i: (i, 0)),
               pl.BlockSpec((BM, HH), lambda i: (i, 0)))
  return pl.pallas_call(
      functools.partial(_mm_kernel, pre),
      grid=(NBLK,),
      in_specs=in_specs,
      out_specs=out_specs,
      out_shape=(jax.ShapeDtypeStruct((NP, HH), _f32),
                 jax.ShapeDtypeStruct((NP, HH), _f32)),
  )(*args)


GP = 25
NG = CH_D // GP
NB = 4


def _agg_half(y2, out2, s, src4, dst4, src_blk, dst_blk, rows, gsem, ssem,
              acc):
  mine = pl.ds(s * RT, RT)
  pltpu.sync_copy(y2.at[mine], acc.at[mine])
  plsc.subcore_barrier()

  def gather(u):
    b = u % NB
    return pltpu.async_copy(y2.at[src_blk.at[u]], rows[b], gsem[b])

  for half in range(CH_A // CH_D):

    def group(g, carry):
      pltpu.sync_copy(src4.at[s, half, g], src_blk)
      pltpu.sync_copy(dst4.at[s, half, g], dst_blk)
      gd = [None] * GP
      sd = [None] * GP
      for u in range(NB - 1):
        gd[u] = gather(u)
      for u in range(GP):
        b = u % NB
        gd[u].wait()
        if u >= 1:
          sd[u - 1].wait()
        if u + NB - 1 < GP:
          gd[u + NB - 1] = gather(u + NB - 1)
        sd[u] = pltpu.async_copy(
            rows[b], acc.at[dst_blk.at[u]], ssem[b], add=True)
      sd[GP - 1].wait()
      return carry

    lax.fori_loop(0, NG, group, 0)

  plsc.subcore_barrier()
  pltpu.sync_copy(acc.at[mine], out2.at[mine])


def _agg_body(y0, y1, src4, dst4, a0, a1, src_blk, dst_blk, *rest):
  rows = rest[:NB]
  gsem = rest[NB:2 * NB]
  ssem = rest[2 * NB:3 * NB]
  acc = rest[3 * NB]
  c = lax.axis_index("c")
  s = lax.axis_index("s")

  @pl.when(c == 0)
  def _():
    _agg_half(y0, a0, s, src4, dst4, src_blk, dst_blk, rows, gsem, ssem, acc)

  @pl.when(c == 1)
  def _():
    _agg_half(y1, a1, s, src4, dst4, src_blk, dst_blk, rows, gsem, ssem, acc)


_agg_call = pl.kernel(
    _agg_body,
    out_type=(jax.ShapeDtypeStruct((NP, HH), _f32),
              jax.ShapeDtypeStruct((NP, HH), _f32)),
    mesh=_sc_mesh,
    scratch_types=(
        [pltpu.VMEM((GP, K), jnp.int32),
         pltpu.VMEM((GP, K), jnp.int32)] +
        [pltpu.VMEM((K, HH), _f32)] * NB +
        [pltpu.SemaphoreType.DMA] * (2 * NB) +
        [pltpu.VMEM_SHARED((NP, HH), _f32)]
    ),
)


def _pool_head_kernel(a0_ref, a1_ref, dinv_ref, b3_ref, batch_ref, st_ref,
                      wl_ref, bl_ref,
                      wf1a_ref, wf1b_ref, bf1_ref, wf2_ref, bf2_ref,
                      wf3_ref, bf3_ref, wf4_ref, bf4_ref, wf5_ref, bf5_ref,
                      o_ref, acc_p, acc_c):
  i = pl.program_id(0)

  @pl.when(i == 0)
  def _():
    acc_p[...] = jnp.zeros_like(acc_p)
    acc_c[...] = jnp.zeros_like(acc_c)

  dinv = dinv_ref[...]
  b3 = b3_ref[...]
  g0 = a0_ref[...] * dinv + b3[:, :HH]
  g1 = a1_ref[...] * dinv + b3[:, HH:]
  ids = lax.broadcasted_iota(jnp.int32, (1, G), 1)
  p = (batch_ref[...] == ids).astype(_f32)
  cdims = (((0,), (0,)), ((), ()))
  acc_p[:, :HH] += lax.dot_general(p, g0, cdims,
                                   preferred_element_type=_f32, precision=lax.Precision.HIGHEST)
  acc_p[:, HH:] += lax.dot_general(p, g1, cdims,
                                   preferred_element_type=_f32, precision=lax.Precision.HIGHEST)
  acc_c[...] += lax.dot_general(p, jnp.ones((BM, 1), _f32), cdims,
                                preferred_element_type=_f32, precision=lax.Precision.HIGHEST)

  @pl.when(i == NBLK - 1)
  def _():
    pooled = acc_p[...] / jnp.maximum(acc_c[...], 1.0)
    z = jnp.dot(pooled, wl_ref[...], preferred_element_type=_f32, precision=lax.Precision.HIGHEST) + bl_ref[...]
    zr = jnp.maximum(z, 0.0)
    sr = jnp.maximum(st_ref[...], 0.0)
    z = jnp.dot(zr, wf1a_ref[...], preferred_element_type=_f32, precision=lax.Precision.HIGHEST)
    z += jnp.dot(sr, wf1b_ref[...], preferred_element_type=_f32, precision=lax.Precision.HIGHEST)
    z = jnp.maximum(z + bf1_ref[...], 0.0)
    z = jnp.maximum(
        jnp.dot(z, wf2_ref[...], preferred_element_type=_f32, precision=lax.Precision.HIGHEST) + bf2_ref[...],
        0.0)
    z = jnp.maximum(
        jnp.dot(z, wf3_ref[...], preferred_element_type=_f32, precision=lax.Precision.HIGHEST) + bf3_ref[...],
        0.0)
    z = jnp.maximum(
        jnp.dot(z, wf4_ref[...], preferred_element_type=_f32, precision=lax.Precision.HIGHEST) + bf4_ref[...],
        0.0)
    o_ref[...] = (
        jnp.dot(z, wf5_ref[...], preferred_element_type=_f32, precision=lax.Precision.HIGHEST) + bf5_ref[...])


def _pool_head_call(a0, a1, dinv_n1, b3, batch_n1, st, wl, bl, wf1a, wf1b,
                    bf1, wf2, bf2, wf3, bf3, wf4, bf4, wf5, bf5):
  full = lambda a: pl.BlockSpec(a.shape, lambda i: tuple(0 for _ in a.shape))
  in_specs = [
      pl.BlockSpec((BM, HH), lambda i: (i, 0)),
      pl.BlockSpec((BM, HH), lambda i: (i, 0)),
      pl.BlockSpec((BM, 1), lambda i: (i, 0)),
      pl.BlockSpec((1, H), lambda i: (0, 0)),
      pl.BlockSpec((BM, 1), lambda i: (i, 0)),
      full(st), full(wl), full(bl), full(wf1a), full(wf1b), full(bf1),
      full(wf2), full(bf2), full(wf3), full(bf3), full(wf4), full(bf4),
      full(wf5), full(bf5),
  ]
  return pl.pallas_call(
      _pool_head_kernel,
      grid=(NBLK,),
      in_specs=in_specs,
      out_specs=pl.BlockSpec((G, 1), lambda i: (0, 0)),
      out_shape=jax.ShapeDtypeStruct((G, 1), _f32),
      scratch_shapes=[pltpu.VMEM((G, H), _f32), pltpu.VMEM((G, 1), _f32)],
  )(a0, a1, dinv_n1, b3, batch_n1, st, wl, bl, wf1a, wf1b, bf1, wf2, bf2,
    wf3, bf3, wf4, bf4, wf5, bf5)


def kernel(x, edge_index, batch, static_features, W1, b1, W2, b2, W3, b3,
           Wl, bl, Wf1, bf1, Wf2, bf2, Wf3, bf3, Wf4, bf4, Wf5, bf5):
  src3 = edge_index[0].reshape(NS, CH_A // CH_D, NG, GP, K)
  dst3 = edge_index[1].reshape(NS, CH_A // CH_D, NG, GP, K)
  dst3d = edge_index[1].reshape(NC * NS, CH_D, K)
  zeros_h = jnp.zeros((RT, DW), _f32)
  ones_h = jnp.ones((K, DW), _f32)
  x_p = jnp.pad(x, ((0, NP - N), (0, 0)))
  batch_p = jnp.pad(batch, (0, NP - N), constant_values=G)

  dd = _deg_call(dst3d, zeros_h, ones_h)
  dinv_n1 = _dinv_call(dd[:NP], dd[NP:])

  y0, y1 = _mm_call([x_p], [W1], dinv_n1)
  a0, a1 = _agg_call(y0, y1, src3, dst3)

  y0, y1 = _mm_call([a0, a1], [W2[:HH], W2[HH:]], dinv_n1,
                    b_prev=b1.reshape(1, H))
  a0, a1 = _agg_call(y0, y1, src3, dst3)

  y0, y1 = _mm_call([a0, a1], [W3[:HH], W3[HH:]], dinv_n1,
                    b_prev=b2.reshape(1, H))
  a0, a1 = _agg_call(y0, y1, src3, dst3)

  out = _pool_head_call(
      a0, a1, dinv_n1, b3.reshape(1, H), batch_p.reshape(NP, 1),
      static_features, Wl, bl.reshape(1, 9),
      Wf1[:9], Wf1[9:], bf1.reshape(1, 32), Wf2, bf2.reshape(1, 64),
      Wf3, bf3.reshape(1, 32), Wf4, bf4.reshape(1, 16), Wf5,
      bf5.reshape(1, 1))
  return out.reshape(G)
